# baseline (speedup 1.0000x reference)
import jax
import jax.numpy as jnp
from jax.experimental import pallas as pl
from jax.experimental.pallas import tpu as pltpu

BC = 8


def _leaky(v):
    return jnp.where(v >= 0, v, 0.2 * v)


def _mm(a, b):
    return jax.lax.dot_general(a.astype(jnp.bfloat16), b.astype(jnp.bfloat16),
                               (((1,), (0,)), ((), ())),
                               preferred_element_type=jnp.float32)


def _mmf(a, b):
    return jax.lax.dot_general(a, b, (((1,), (0,)), ((), ())),
                               preferred_element_type=jnp.float32)


def _mega_body(x0, x1, x2, x3, W1s, b1, W2s, b2, Wfi, bfi, Wfo, bfo,
               Es, ETs, e2s, Wd1s, bd1, Wd2s, bd2,
               y_ref, loss_ref, cnt_ref):
    bc = x0.shape[0]
    t1h = 128
    rows = bc * t1h

    a0 = x0[...]
    a1 = x1[...]
    a2 = x2[...]
    a3 = x3[...]
    cin = a0.shape[2]

    def win(a, s):
        return a[:, s:s + t1h, :].reshape(rows, cin)

    W10, W11, W12, W13 = W1s[0], W1s[1], W1s[2], W1s[3]
    he = _mm(win(a0, 0), W10) + _mm(win(a1, 0), W11) \
        + _mm(win(a2, 0), W12) + _mm(win(a3, 0), W13)
    ho = _mm(win(a2, 0), W10) + _mm(win(a3, 0), W11) \
        + _mm(win(a0, 1), W12) + _mm(win(a1, 1), W13)
    he = _leaky(he + b1[...])
    ho = _leaky(ho + b1[...])

    H = he.shape[1]
    he3 = he.reshape(bc, t1h, H)
    ho3 = ho.reshape(bc, t1h, H)
    zrow = jnp.zeros((bc, 1, H), jnp.float32)

    ho_pad = jnp.concatenate([zrow, ho3], axis=1)
    he_pad = jnp.concatenate([he3, zrow], axis=1)
    V0, V1, V2, V3 = W2s[0], W2s[1], W2s[2], W2s[3]
    h = _mm(ho_pad[:, 0:t1h].reshape(rows, H), V0) \
        + _mm(he, V1) + _mm(ho, V2) \
        + _mm(he_pad[:, 1:t1h + 1].reshape(rows, H), V3)
    h = _leaky(h + b2[...])

    zt = _mm(h, Wfi[...]) + bfi[...]
    zh = jnp.round(zt)
    z_fsq = _mm(zh, Wfo[...]) + bfo[...]

    r = h - z_fsq
    loss_acc = jnp.float32(0.0)
    ncode = Es.shape[1]
    code_iota = jax.lax.broadcasted_iota(jnp.int32, (rows, ncode), 1)
    ones_row = jnp.ones((1, rows), jnp.float32)
    for l in range(4):
        El = Es[l]
        ElT = ETs[l]
        e2 = e2s[l]
        G = _mm(r, ElT)
        rn = jnp.sum(r * r, axis=1)
        score = (rn[:, None] + e2) - 2.0 * G
        m = jnp.min(score, axis=1)
        idx = jnp.min(jnp.where(score == m[:, None], code_iota, ncode),
                      axis=1)
        mask = (code_iota == idx[:, None]).astype(jnp.float32)
        q = _mmf(mask, El)
        loss_acc = loss_acc + jnp.sum(m)
        cnt_ref[:, l, :] = _mmf(ones_row, mask)
        r = r - q

    loss_ref[...] = jnp.broadcast_to(loss_acc.reshape(1, 1), loss_ref.shape)

    z3 = (h - r).reshape(bc, t1h, H)
    zs = jnp.concatenate([zrow, z3, zrow], axis=1)
    D0, D1, D2, D3 = Wd1s[0], Wd1s[1], Wd1s[2], Wd1s[3]
    y1e = _leaky(_mm(zs[:, 1:t1h + 1].reshape(rows, H), D1)
                 + _mm(zs[:, 0:t1h].reshape(rows, H), D3) + bd1[...])
    y1o = _leaky(_mm(zs[:, 2:t1h + 2].reshape(rows, H), D0)
                 + _mm(zs[:, 1:t1h + 1].reshape(rows, H), D2) + bd1[...])

    U0, U1, U2, U3 = Wd2s[0], Wd2s[1], Wd2s[2], Wd2s[3]
    y1e3 = y1e.reshape(bc, t1h, H)
    y1o3 = y1o.reshape(bc, t1h, H)
    y1e_s = jnp.concatenate([y1e3, zrow], axis=1)
    y1o_s = jnp.concatenate([zrow, y1o3], axis=1)
    cout = U0.shape[1]
    bd2v = bd2[...]
    p0 = _mm(y1e, U1) + _mm(y1o_s[:, 0:t1h].reshape(rows, H), U3) + bd2v
    p1 = _mm(y1o, U0) + _mm(y1e, U2) + bd2v
    p2 = _mm(y1o, U1) + _mm(y1e, U3) + bd2v
    p3 = _mm(y1e_s[:, 1:t1h + 1].reshape(rows, H), U0) + _mm(y1o, U2) + bd2v
    y_ref[:, 0, :, :] = p0.reshape(bc, t1h, cout)
    y_ref[:, 1, :, :] = p1.reshape(bc, t1h, cout)
    y_ref[:, 2, :, :] = p2.reshape(bc, t1h, cout)
    y_ref[:, 3, :, :] = p3.reshape(bc, t1h, cout)


def _reduce_body(loss_part, cnt_part, loss_ref, ppl_ref, nrows):
    total = jnp.sum(loss_part[:, :, 0:1])
    hdim = 512.0
    loss_ref[...] = (0.25 * total / (nrows * hdim)).reshape(1, 1)
    counts = jnp.sum(cnt_part[...], axis=0)
    avg = counts * (1.0 / nrows)
    ent = -jnp.sum(avg * jnp.log(avg + 1e-10), axis=1)
    ppl_ref[...] = jnp.mean(jnp.exp(ent)).reshape(1, 1)


def kernel(x, We1, be1, We2, be2, Wfi, bfi, Wfo, bfo,
           E1, E2, E3, E4, Wd1, bd1, Wd2, bd2, interpret=False):
    B, C, T = x.shape
    H = We1.shape[0]
    NE = E1.shape[0]
    F = Wfi.shape[0]
    nch = B // BC
    t2 = T // 4
    nrows = B * t2
    cpad = ((C + 7) // 8) * 8
    FP = 128

    xt = jnp.transpose(x, (0, 2, 1))
    xt = jnp.pad(xt, ((0, 0), (1, 7), (0, cpad - C)))
    xq = xt.reshape(B, (T + 8) // 4, 4, cpad).astype(jnp.bfloat16)
    x0, x1, x2, x3 = (xq[:, :, k, :] for k in range(4))

    bf = jnp.bfloat16
    W1s = jnp.pad(jnp.transpose(We1, (2, 1, 0)),
                  ((0, 0), (0, cpad - C), (0, 0))).astype(bf)
    W2s = jnp.transpose(We2, (2, 1, 0)).astype(bf)
    WfiP = jnp.pad(jnp.transpose(Wfi), ((0, 0), (0, FP - F))).astype(bf)
    bfiP = jnp.pad(bfi, (0, FP - F)).reshape(1, FP)
    WfoP = jnp.pad(jnp.transpose(Wfo), ((0, FP - F), (0, 0))).astype(bf)
    Es = jnp.stack([E1, E2, E3, E4])
    ETs = jnp.transpose(Es, (0, 2, 1)).astype(bf)
    e2s = jnp.sum(Es ** 2, axis=2)[:, None, :]
    Wd1s = jnp.transpose(Wd1, (2, 0, 1)).astype(bf)
    Wd2s = jnp.pad(jnp.transpose(Wd2, (2, 0, 1)),
                   ((0, 0), (0, 0), (0, cpad - C))).astype(bf)

    b1 = be1.reshape(1, H)
    b2 = be2.reshape(1, H)
    bd1r = bd1.reshape(1, H)
    bd2r = jnp.pad(bd2, (0, cpad - C)).reshape(1, cpad)
    bfor = bfo.reshape(1, H)

    t1h = t2
    bspec = lambda shape, imap: pl.BlockSpec(shape, imap)

    in_specs = [
        bspec((BC, t1h + 2, cpad), lambda i: (i, 0, 0)),
        bspec((BC, t1h + 2, cpad), lambda i: (i, 0, 0)),
        bspec((BC, t1h + 2, cpad), lambda i: (i, 0, 0)),
        bspec((BC, t1h + 2, cpad), lambda i: (i, 0, 0)),
        bspec((4, cpad, H), lambda i: (0, 0, 0)),
        bspec((1, H), lambda i: (0, 0)),
        bspec((4, H, H), lambda i: (0, 0, 0)),
        bspec((1, H), lambda i: (0, 0)),
        bspec((H, FP), lambda i: (0, 0)),
        bspec((1, FP), lambda i: (0, 0)),
        bspec((FP, H), lambda i: (0, 0)),
        bspec((1, H), lambda i: (0, 0)),
        bspec((4, NE, H), lambda i: (0, 0, 0)),
        bspec((4, H, NE), lambda i: (0, 0, 0)),
        bspec((4, 1, NE), lambda i: (0, 0, 0)),
        bspec((4, H, H), lambda i: (0, 0, 0)),
        bspec((1, H), lambda i: (0, 0)),
        bspec((4, H, cpad), lambda i: (0, 0, 0)),
        bspec((1, cpad), lambda i: (0, 0)),
    ]
    out_specs = [
        bspec((BC, 4, t1h, cpad), lambda i: (i, 0, 0, 0)),
        bspec((1, 1, 128), lambda i: (i, 0, 0)),
        bspec((1, 4, NE), lambda i: (i, 0, 0)),
    ]
    out_shapes = [
        jax.ShapeDtypeStruct((B, 4, t1h, cpad), jnp.float32),
        jax.ShapeDtypeStruct((nch, 1, 128), jnp.float32),
        jax.ShapeDtypeStruct((nch, 4, NE), jnp.float32),
    ]

    yp, loss_part, cnt_part = pl.pallas_call(
        _mega_body,
        grid=(nch,),
        in_specs=in_specs,
        out_specs=out_specs,
        out_shape=out_shapes,
        compiler_params=pltpu.CompilerParams(
            dimension_semantics=("parallel",)),
        interpret=interpret,
    )(x0, x1, x2, x3, W1s, b1, W2s, b2, WfiP, bfiP, WfoP, bfor,
      Es, ETs, e2s, Wd1s, bd1r, Wd2s, bd2r)

    import functools
    loss2, ppl2 = pl.pallas_call(
        functools.partial(_reduce_body, nrows=float(nrows)),
        in_specs=[bspec((nch, 1, 128), lambda: (0, 0, 0)),
                  bspec((nch, 4, NE), lambda: (0, 0, 0))],
        out_specs=[bspec((1, 1), lambda: (0, 0)),
                   bspec((1, 1), lambda: (0, 0))],
        out_shape=[jax.ShapeDtypeStruct((1, 1), jnp.float32),
                   jax.ShapeDtypeStruct((1, 1), jnp.float32)],
        interpret=interpret,
    )(loss_part, cnt_part)

    y = jnp.transpose(yp, (0, 2, 1, 3)).reshape(B, T, cpad)[:, :, :C]
    y = jnp.transpose(y, (0, 2, 1))
    return (y, loss2.reshape(()), ppl2.reshape(()))


# speedup vs baseline: 1.0234x; 1.0234x over previous
import jax
import jax.numpy as jnp
from jax.experimental import pallas as pl
from jax.experimental.pallas import tpu as pltpu

BC = 8


def _leaky(v):
    return jnp.where(v >= 0, v, 0.2 * v)


def _mm(a, b):
    return jax.lax.dot_general(a.astype(jnp.bfloat16), b.astype(jnp.bfloat16),
                               (((1,), (0,)), ((), ())),
                               preferred_element_type=jnp.float32)


def _mmf(a, b):
    return jax.lax.dot_general(a, b, (((1,), (0,)), ((), ())),
                               preferred_element_type=jnp.float32)


def _mega_body(x0, x1, x2, x3, W1s, b1, W2s, b2, Wfi, bfi, Wfo, bfo,
               Es, ETs, e2s, Wd1s, bd1, Wd2s, bd2,
               y_ref, loss_ref, cnt_ref):
    bc = x0.shape[0]
    t1h = 128
    rows = bc * t1h

    a0 = x0[...]
    a1 = x1[...]
    a2 = x2[...]
    a3 = x3[...]
    cin = a0.shape[2]

    def win(a, s):
        return a[:, s:s + t1h, :].reshape(rows, cin)

    W10, W11, W12, W13 = W1s[0], W1s[1], W1s[2], W1s[3]
    he = _mm(win(a0, 0), W10) + _mm(win(a1, 0), W11) \
        + _mm(win(a2, 0), W12) + _mm(win(a3, 0), W13)
    ho = _mm(win(a2, 0), W10) + _mm(win(a3, 0), W11) \
        + _mm(win(a0, 1), W12) + _mm(win(a1, 1), W13)
    he = _leaky(he + b1[...])
    ho = _leaky(ho + b1[...])

    H = he.shape[1]
    he3 = he.reshape(bc, t1h, H)
    ho3 = ho.reshape(bc, t1h, H)
    zrow = jnp.zeros((bc, 1, H), jnp.float32)

    ho_pad = jnp.concatenate([zrow, ho3], axis=1)
    he_pad = jnp.concatenate([he3, zrow], axis=1)
    V0, V1, V2, V3 = W2s[0], W2s[1], W2s[2], W2s[3]
    h = _mm(ho_pad[:, 0:t1h].reshape(rows, H), V0) \
        + _mm(he, V1) + _mm(ho, V2) \
        + _mm(he_pad[:, 1:t1h + 1].reshape(rows, H), V3)
    h = _leaky(h + b2[...])

    zt = _mm(h, Wfi[...]) + bfi[...]
    zh = jnp.round(zt)
    z_fsq = _mm(zh, Wfo[...]) + bfo[...]

    r = h - z_fsq
    loss_acc = jnp.float32(0.0)
    ncode = Es.shape[1]
    code_iota = jax.lax.broadcasted_iota(jnp.int32, (rows, ncode), 1)
    ones_row = jnp.ones((1, rows), jnp.float32)
    for l in range(4):
        El = Es[l]
        ElT = ETs[l]
        e2 = e2s[l]
        G = _mm(r, ElT)
        rn = jnp.sum(r * r, axis=1)
        score = (rn[:, None] + e2) - 2.0 * G
        m = jnp.min(score, axis=1)
        idx = jnp.min(jnp.where(score == m[:, None], code_iota, ncode),
                      axis=1)
        mask = (code_iota == idx[:, None]).astype(jnp.float32)
        q = _mmf(mask, El)
        loss_acc = loss_acc + jnp.sum(m)
        cnt_ref[:, l, :] = jnp.sum(mask, axis=0, keepdims=True)
        r = r - q

    loss_ref[...] = jnp.broadcast_to(loss_acc.reshape(1, 1), loss_ref.shape)

    z3 = (h - r).reshape(bc, t1h, H)
    zs = jnp.concatenate([zrow, z3, zrow], axis=1)
    D0, D1, D2, D3 = Wd1s[0], Wd1s[1], Wd1s[2], Wd1s[3]
    y1e = _leaky(_mm(zs[:, 1:t1h + 1].reshape(rows, H), D1)
                 + _mm(zs[:, 0:t1h].reshape(rows, H), D3) + bd1[...])
    y1o = _leaky(_mm(zs[:, 2:t1h + 2].reshape(rows, H), D0)
                 + _mm(zs[:, 1:t1h + 1].reshape(rows, H), D2) + bd1[...])

    U0, U1, U2, U3 = Wd2s[0], Wd2s[1], Wd2s[2], Wd2s[3]
    y1e3 = y1e.reshape(bc, t1h, H)
    y1o3 = y1o.reshape(bc, t1h, H)
    y1e_s = jnp.concatenate([y1e3, zrow], axis=1)
    y1o_s = jnp.concatenate([zrow, y1o3], axis=1)
    cout = U0.shape[1]
    bd2v = bd2[...]
    p0 = _mm(y1e, U1) + _mm(y1o_s[:, 0:t1h].reshape(rows, H), U3) + bd2v
    p1 = _mm(y1o, U0) + _mm(y1e, U2) + bd2v
    p2 = _mm(y1o, U1) + _mm(y1e, U3) + bd2v
    p3 = _mm(y1e_s[:, 1:t1h + 1].reshape(rows, H), U0) + _mm(y1o, U2) + bd2v
    y_ref[:, 0, :, :] = p0.reshape(bc, t1h, cout)
    y_ref[:, 1, :, :] = p1.reshape(bc, t1h, cout)
    y_ref[:, 2, :, :] = p2.reshape(bc, t1h, cout)
    y_ref[:, 3, :, :] = p3.reshape(bc, t1h, cout)


def _reduce_body(loss_part, cnt_part, loss_ref, ppl_ref, nrows):
    total = jnp.sum(loss_part[:, :, 0:1])
    hdim = 512.0
    loss_ref[...] = (0.25 * total / (nrows * hdim)).reshape(1, 1)
    counts = jnp.sum(cnt_part[...], axis=0)
    avg = counts * (1.0 / nrows)
    ent = -jnp.sum(avg * jnp.log(avg + 1e-10), axis=1)
    ppl_ref[...] = jnp.mean(jnp.exp(ent)).reshape(1, 1)


def kernel(x, We1, be1, We2, be2, Wfi, bfi, Wfo, bfo,
           E1, E2, E3, E4, Wd1, bd1, Wd2, bd2, interpret=False):
    B, C, T = x.shape
    H = We1.shape[0]
    NE = E1.shape[0]
    F = Wfi.shape[0]
    nch = B // BC
    t2 = T // 4
    nrows = B * t2
    cpad = ((C + 7) // 8) * 8
    FP = 128

    xt = jnp.transpose(x, (0, 2, 1))
    xt = jnp.pad(xt, ((0, 0), (1, 7), (0, cpad - C)))
    xq = xt.reshape(B, (T + 8) // 4, 4, cpad).astype(jnp.bfloat16)
    x0, x1, x2, x3 = (xq[:, :, k, :] for k in range(4))

    bf = jnp.bfloat16
    W1s = jnp.pad(jnp.transpose(We1, (2, 1, 0)),
                  ((0, 0), (0, cpad - C), (0, 0))).astype(bf)
    W2s = jnp.transpose(We2, (2, 1, 0)).astype(bf)
    WfiP = jnp.pad(jnp.transpose(Wfi), ((0, 0), (0, FP - F))).astype(bf)
    bfiP = jnp.pad(bfi, (0, FP - F)).reshape(1, FP)
    WfoP = jnp.pad(jnp.transpose(Wfo), ((0, FP - F), (0, 0))).astype(bf)
    Es = jnp.stack([E1, E2, E3, E4])
    ETs = jnp.transpose(Es, (0, 2, 1)).astype(bf)
    e2s = jnp.sum(Es ** 2, axis=2)[:, None, :]
    Wd1s = jnp.transpose(Wd1, (2, 0, 1)).astype(bf)
    Wd2s = jnp.pad(jnp.transpose(Wd2, (2, 0, 1)),
                   ((0, 0), (0, 0), (0, cpad - C))).astype(bf)

    b1 = be1.reshape(1, H)
    b2 = be2.reshape(1, H)
    bd1r = bd1.reshape(1, H)
    bd2r = jnp.pad(bd2, (0, cpad - C)).reshape(1, cpad)
    bfor = bfo.reshape(1, H)

    t1h = t2
    bspec = lambda shape, imap: pl.BlockSpec(shape, imap)

    in_specs = [
        bspec((BC, t1h + 2, cpad), lambda i: (i, 0, 0)),
        bspec((BC, t1h + 2, cpad), lambda i: (i, 0, 0)),
        bspec((BC, t1h + 2, cpad), lambda i: (i, 0, 0)),
        bspec((BC, t1h + 2, cpad), lambda i: (i, 0, 0)),
        bspec((4, cpad, H), lambda i: (0, 0, 0)),
        bspec((1, H), lambda i: (0, 0)),
        bspec((4, H, H), lambda i: (0, 0, 0)),
        bspec((1, H), lambda i: (0, 0)),
        bspec((H, FP), lambda i: (0, 0)),
        bspec((1, FP), lambda i: (0, 0)),
        bspec((FP, H), lambda i: (0, 0)),
        bspec((1, H), lambda i: (0, 0)),
        bspec((4, NE, H), lambda i: (0, 0, 0)),
        bspec((4, H, NE), lambda i: (0, 0, 0)),
        bspec((4, 1, NE), lambda i: (0, 0, 0)),
        bspec((4, H, H), lambda i: (0, 0, 0)),
        bspec((1, H), lambda i: (0, 0)),
        bspec((4, H, cpad), lambda i: (0, 0, 0)),
        bspec((1, cpad), lambda i: (0, 0)),
    ]
    out_specs = [
        bspec((BC, 4, t1h, cpad), lambda i: (i, 0, 0, 0)),
        bspec((1, 1, 128), lambda i: (i, 0, 0)),
        bspec((1, 4, NE), lambda i: (i, 0, 0)),
    ]
    out_shapes = [
        jax.ShapeDtypeStruct((B, 4, t1h, cpad), jnp.float32),
        jax.ShapeDtypeStruct((nch, 1, 128), jnp.float32),
        jax.ShapeDtypeStruct((nch, 4, NE), jnp.float32),
    ]

    yp, loss_part, cnt_part = pl.pallas_call(
        _mega_body,
        grid=(nch,),
        in_specs=in_specs,
        out_specs=out_specs,
        out_shape=out_shapes,
        compiler_params=pltpu.CompilerParams(
            dimension_semantics=("parallel",)),
        interpret=interpret,
    )(x0, x1, x2, x3, W1s, b1, W2s, b2, WfiP, bfiP, WfoP, bfor,
      Es, ETs, e2s, Wd1s, bd1r, Wd2s, bd2r)

    import functools
    loss2, ppl2 = pl.pallas_call(
        functools.partial(_reduce_body, nrows=float(nrows)),
        in_specs=[bspec((nch, 1, 128), lambda: (0, 0, 0)),
                  bspec((nch, 4, NE), lambda: (0, 0, 0))],
        out_specs=[bspec((1, 1), lambda: (0, 0)),
                   bspec((1, 1), lambda: (0, 0))],
        out_shape=[jax.ShapeDtypeStruct((1, 1), jnp.float32),
                   jax.ShapeDtypeStruct((1, 1), jnp.float32)],
        interpret=interpret,
    )(loss_part, cnt_part)

    y = jnp.transpose(yp, (0, 2, 1, 3)).reshape(B, T, cpad)[:, :, :C]
    y = jnp.transpose(y, (0, 2, 1))
    return (y, loss2.reshape(()), ppl2.reshape(()))
